# baseline (device time: 182660 ns/iter reference)
import jax
import jax.numpy as jnp
from jax import lax
from jax.experimental import pallas as pl
from jax.experimental.pallas import tpu as pltpu

N_DEV = 8


def kernel(x, router_W, route_idx, expert_W, shared_W):
    n_tok, d = x.shape
    e_loc, _, h = expert_W.shape

    def body(x_ref, rW_ref, idx_ref, eW_ref, sW_ref, out_ref,
             chunks_ref, send_sems, recv_sems):
        my = lax.axis_index("i")
        left = lax.rem(my + N_DEV - 1, N_DEV)
        right = lax.rem(my + 1, N_DEV)

        chunks_ref[0] = eW_ref[...]

        barrier_sem = pltpu.get_barrier_semaphore()
        for nbr in (left, right):
            pl.semaphore_signal(
                barrier_sem, inc=1,
                device_id=(nbr,), device_id_type=pl.DeviceIdType.MESH,
            )
        pl.semaphore_wait(barrier_sem, 2)

        xv = x_ref[...]
        idx = idx_ref[...]

        scores = jnp.dot(xv, rW_ref[...], preferred_element_type=jnp.float32)
        smax = jnp.max(scores, axis=1, keepdims=True)
        probs = jnp.exp(scores - smax)
        probs = probs / jnp.sum(probs, axis=1, keepdims=True)
        eids = lax.broadcasted_iota(jnp.int32, scores.shape, 1)
        p = jnp.sum(jnp.where(eids == idx, probs, 0.0), axis=1, keepdims=True)

        acc = jnp.dot(xv, sW_ref[...], preferred_element_type=jnp.float32)

        def add_chunk(acc, slot, origin):
            for j in range(e_loc):
                e = origin * e_loc + j
                gate = p * (idx == e).astype(jnp.float32)
                acc = acc + jnp.dot(
                    xv * gate, chunks_ref[slot, j],
                    preferred_element_type=jnp.float32,
                )
            return acc

        for hop in range(N_DEV - 1):
            rdma = pltpu.make_async_remote_copy(
                src_ref=chunks_ref.at[hop],
                dst_ref=chunks_ref.at[hop + 1],
                send_sem=send_sems.at[hop],
                recv_sem=recv_sems.at[hop],
                device_id=(right,),
                device_id_type=pl.DeviceIdType.MESH,
            )
            rdma.start()
            acc = add_chunk(acc, hop, lax.rem(my + N_DEV - hop, N_DEV))
            rdma.wait()

        acc = add_chunk(acc, N_DEV - 1, lax.rem(my + 1, N_DEV))
        out_ref[...] = acc

    return pl.pallas_call(
        body,
        out_shape=jax.ShapeDtypeStruct((n_tok, h), jnp.float32),
        in_specs=[pl.BlockSpec(memory_space=pltpu.VMEM)] * 5,
        out_specs=pl.BlockSpec(memory_space=pltpu.VMEM),
        scratch_shapes=[
            pltpu.VMEM((N_DEV, e_loc, d, h), jnp.float32),
            pltpu.SemaphoreType.DMA((N_DEV - 1,)),
            pltpu.SemaphoreType.DMA((N_DEV - 1,)),
        ],
        compiler_params=pltpu.CompilerParams(collective_id=0),
    )(x, router_W, route_idx, expert_W, shared_W)


# device time: 109442 ns/iter; 1.6690x vs baseline; 1.6690x over previous
import jax
import jax.numpy as jnp
from jax import lax
from jax.experimental import pallas as pl
from jax.experimental.pallas import tpu as pltpu

N_DEV = 8
R_HOPS = 4
L_HOPS = 3


def kernel(x, router_W, route_idx, expert_W, shared_W):
    n_tok, d = x.shape
    e_loc, _, h = expert_W.shape

    def body(x_ref, rW_ref, idx_ref, eW_ref, sW_ref, out_ref,
             chunks_ref, r_send, r_recv, l_send, l_recv):
        my = lax.axis_index("i")
        left = lax.rem(my + N_DEV - 1, N_DEV)
        right = lax.rem(my + 1, N_DEV)

        chunks_ref[0] = eW_ref[...]

        barrier_sem = pltpu.get_barrier_semaphore()
        for nbr in (left, right):
            pl.semaphore_signal(
                barrier_sem, inc=1,
                device_id=(nbr,), device_id_type=pl.DeviceIdType.MESH,
            )
        pl.semaphore_wait(barrier_sem, 2)

        def send_right(hop):
            rdma = pltpu.make_async_remote_copy(
                src_ref=chunks_ref.at[0 if hop == 0 else hop],
                dst_ref=chunks_ref.at[hop + 1],
                send_sem=r_send.at[hop],
                recv_sem=r_recv.at[hop],
                device_id=(right,),
                device_id_type=pl.DeviceIdType.MESH,
            )
            rdma.start()
            return rdma

        def send_left(hop):
            rdma = pltpu.make_async_remote_copy(
                src_ref=chunks_ref.at[0 if hop == 0 else 4 + hop],
                dst_ref=chunks_ref.at[5 + hop],
                send_sem=l_send.at[hop],
                recv_sem=l_recv.at[hop],
                device_id=(left,),
                device_id_type=pl.DeviceIdType.MESH,
            )
            rdma.start()
            return rdma

        xv = x_ref[...]
        idx = idx_ref[...]

        r_rdma = send_right(0)
        l_rdma = send_left(0)

        scores = jnp.dot(xv, rW_ref[...], preferred_element_type=jnp.float32)
        smax = jnp.max(scores, axis=1, keepdims=True)
        probs = jnp.exp(scores - smax)
        probs = probs / jnp.sum(probs, axis=1, keepdims=True)
        eids = lax.broadcasted_iota(jnp.int32, scores.shape, 1)
        p = jnp.sum(jnp.where(eids == idx, probs, 0.0), axis=1, keepdims=True)

        acc = jnp.dot(xv, sW_ref[...], preferred_element_type=jnp.float32)

        def add_chunk(acc, slot, origin):
            for j in range(e_loc):
                e = origin * e_loc + j
                gate = p * (idx == e).astype(jnp.float32)
                acc = acc + jnp.dot(
                    xv * gate, chunks_ref[slot, j],
                    preferred_element_type=jnp.float32,
                )
            return acc

        acc = add_chunk(acc, 0, my)

        for hop in range(R_HOPS):
            r_rdma.wait()
            if hop + 1 < R_HOPS:
                r_rdma = send_right(hop + 1)
            acc = add_chunk(acc, hop + 1, lax.rem(my + N_DEV - hop - 1, N_DEV))

            if hop < L_HOPS:
                l_rdma.wait()
                if hop + 1 < L_HOPS:
                    l_rdma = send_left(hop + 1)
                acc = add_chunk(acc, 5 + hop, lax.rem(my + hop + 1, N_DEV))

        out_ref[...] = acc

    return pl.pallas_call(
        body,
        out_shape=jax.ShapeDtypeStruct((n_tok, h), jnp.float32),
        in_specs=[pl.BlockSpec(memory_space=pltpu.VMEM)] * 5,
        out_specs=pl.BlockSpec(memory_space=pltpu.VMEM),
        scratch_shapes=[
            pltpu.VMEM((N_DEV, e_loc, d, h), jnp.float32),
            pltpu.SemaphoreType.DMA((R_HOPS,)),
            pltpu.SemaphoreType.DMA((R_HOPS,)),
            pltpu.SemaphoreType.DMA((L_HOPS,)),
            pltpu.SemaphoreType.DMA((L_HOPS,)),
        ],
        compiler_params=pltpu.CompilerParams(collective_id=0),
    )(x, router_W, route_idx, expert_W, shared_W)


# device time: 64451 ns/iter; 2.8341x vs baseline; 1.6981x over previous
import jax
import jax.numpy as jnp
from jax import lax
from jax.experimental import pallas as pl
from jax.experimental.pallas import tpu as pltpu

N_DEV = 8
R_HOPS = 4
L_HOPS = 3


def kernel(x, router_W, route_idx, expert_W, shared_W):
    n_tok, d = x.shape
    e_loc, _, h = expert_W.shape

    def body(x_ref, rW_ref, idx_ref, eW_ref, sW_ref, out_ref,
             chunks_ref, r_send, r_recv, l_send, l_recv):
        my = lax.axis_index("i")
        left = lax.rem(my + N_DEV - 1, N_DEV)
        right = lax.rem(my + 1, N_DEV)

        chunks_ref[0] = eW_ref[...].astype(jnp.bfloat16)

        barrier_sem = pltpu.get_barrier_semaphore()
        for nbr in (left, right):
            pl.semaphore_signal(
                barrier_sem, inc=1,
                device_id=(nbr,), device_id_type=pl.DeviceIdType.MESH,
            )
        pl.semaphore_wait(barrier_sem, 2)

        def send_right(hop):
            rdma = pltpu.make_async_remote_copy(
                src_ref=chunks_ref.at[0 if hop == 0 else hop],
                dst_ref=chunks_ref.at[hop + 1],
                send_sem=r_send.at[hop],
                recv_sem=r_recv.at[hop],
                device_id=(right,),
                device_id_type=pl.DeviceIdType.MESH,
            )
            rdma.start()
            return rdma

        def send_left(hop):
            rdma = pltpu.make_async_remote_copy(
                src_ref=chunks_ref.at[0 if hop == 0 else 4 + hop],
                dst_ref=chunks_ref.at[5 + hop],
                send_sem=l_send.at[hop],
                recv_sem=l_recv.at[hop],
                device_id=(left,),
                device_id_type=pl.DeviceIdType.MESH,
            )
            rdma.start()
            return rdma

        xv = x_ref[...]
        xb = xv.astype(jnp.bfloat16)
        idx = idx_ref[...]

        r_rdma = send_right(0)
        l_rdma = send_left(0)

        scores = jnp.dot(xv, rW_ref[...], preferred_element_type=jnp.float32)
        smax = jnp.max(scores, axis=1, keepdims=True)
        probs = jnp.exp(scores - smax)
        probs = probs / jnp.sum(probs, axis=1, keepdims=True)
        eids = lax.broadcasted_iota(jnp.int32, scores.shape, 1)
        p = jnp.sum(jnp.where(eids == idx, probs, 0.0), axis=1, keepdims=True)

        acc = jnp.dot(xv, sW_ref[...], preferred_element_type=jnp.float32)

        def add_chunk(acc, slot, origin):
            for j in range(e_loc):
                e = origin * e_loc + j
                gate = (p * (idx == e).astype(jnp.float32)).astype(jnp.bfloat16)
                acc = acc + jnp.dot(
                    xb * gate, chunks_ref[slot, j],
                    preferred_element_type=jnp.float32,
                )
            return acc

        acc = add_chunk(acc, 0, my)

        for hop in range(R_HOPS):
            r_rdma.wait()
            if hop + 1 < R_HOPS:
                r_rdma = send_right(hop + 1)
            acc = add_chunk(acc, hop + 1, lax.rem(my + N_DEV - hop - 1, N_DEV))

            if hop < L_HOPS:
                l_rdma.wait()
                if hop + 1 < L_HOPS:
                    l_rdma = send_left(hop + 1)
                acc = add_chunk(acc, 5 + hop, lax.rem(my + hop + 1, N_DEV))

        out_ref[...] = acc

    return pl.pallas_call(
        body,
        out_shape=jax.ShapeDtypeStruct((n_tok, h), jnp.float32),
        in_specs=[pl.BlockSpec(memory_space=pltpu.VMEM)] * 5,
        out_specs=pl.BlockSpec(memory_space=pltpu.VMEM),
        scratch_shapes=[
            pltpu.VMEM((N_DEV, e_loc, d, h), jnp.bfloat16),
            pltpu.SemaphoreType.DMA((R_HOPS,)),
            pltpu.SemaphoreType.DMA((R_HOPS,)),
            pltpu.SemaphoreType.DMA((L_HOPS,)),
            pltpu.SemaphoreType.DMA((L_HOPS,)),
        ],
        compiler_params=pltpu.CompilerParams(collective_id=0),
    )(x, router_W, route_idx, expert_W, shared_W)


# device time: 52266 ns/iter; 3.4948x vs baseline; 1.2331x over previous
import jax
import jax.numpy as jnp
from jax import lax
from jax.experimental import pallas as pl
from jax.experimental.pallas import tpu as pltpu

N_DEV = 8

_SENDS = (
    (0, 0, 1, 1, -1),
    (1, 0, 3, 3, -1),
    (2, 0, 4, 4, -1),
    (3, 1, 2, 3, 1),
    (4, 3, 7, 4, 3),
    (5, 4, 5, 1, 4),
    (6, 7, 6, 1, 7),
)
_RECV_SLOT = {0: 1, 1: 3, 2: 4, 3: 2, 4: 7, 5: 5, 6: 6}
_WAIT_ORDER = (0, 1, 2, 4, 5, 3, 6)


def kernel(x, router_W, route_idx, expert_W, shared_W):
    n_tok, d = x.shape
    e_loc, _, h = expert_W.shape

    def body(x_ref, rW_ref, idx_ref, eW_ref, sW_ref, out_ref,
             chunks_ref, send_sems, recv_sems):
        my = lax.axis_index("i")
        nbr = {m: jnp.bitwise_xor(my, m) for m in (1, 3, 4)}

        chunks_ref[0] = eW_ref[...].astype(jnp.bfloat16)

        barrier_sem = pltpu.get_barrier_semaphore()
        for m in (1, 3, 4):
            pl.semaphore_signal(
                barrier_sem, inc=1,
                device_id=(nbr[m],), device_id_type=pl.DeviceIdType.MESH,
            )
        pl.semaphore_wait(barrier_sem, 3)

        def make_send(k):
            _, src, dst, m, _ = _SENDS[k]
            return pltpu.make_async_remote_copy(
                src_ref=chunks_ref.at[src],
                dst_ref=chunks_ref.at[dst],
                send_sem=send_sems.at[k],
                recv_sem=recv_sems.at[k],
                device_id=(nbr[m],),
                device_id_type=pl.DeviceIdType.MESH,
            )

        rdmas = {}
        for k in (0, 1, 2):
            rdmas[k] = make_send(k)
            rdmas[k].start()

        xv = x_ref[...]
        xb = xv.astype(jnp.bfloat16)
        idx = idx_ref[...]

        scores = jnp.dot(xv, rW_ref[...], preferred_element_type=jnp.float32)
        smax = jnp.max(scores, axis=1, keepdims=True)
        probs = jnp.exp(scores - smax)
        probs = probs / jnp.sum(probs, axis=1, keepdims=True)
        eids = lax.broadcasted_iota(jnp.int32, scores.shape, 1)
        p = jnp.sum(jnp.where(eids == idx, probs, 0.0), axis=1, keepdims=True)

        acc = jnp.dot(xv, sW_ref[...], preferred_element_type=jnp.float32)

        def add_chunk(acc, slot):
            origin = jnp.bitwise_xor(my, slot)
            for j in range(e_loc):
                e = origin * e_loc + j
                gate = (p * (idx == e).astype(jnp.float32)).astype(jnp.bfloat16)
                acc = acc + jnp.dot(
                    xb * gate, chunks_ref[slot, j],
                    preferred_element_type=jnp.float32,
                )
            return acc

        acc = add_chunk(acc, 0)

        fwd_after = {s[4]: s[0] for s in _SENDS if s[4] >= 0}

        for k in _WAIT_ORDER:
            rdmas[k].wait()
            slot = _RECV_SLOT[k]
            if slot in fwd_after:
                kk = fwd_after[slot]
                rdmas[kk] = make_send(kk)
                rdmas[kk].start()
            acc = add_chunk(acc, slot)

        out_ref[...] = acc

    return pl.pallas_call(
        body,
        out_shape=jax.ShapeDtypeStruct((n_tok, h), jnp.float32),
        in_specs=[pl.BlockSpec(memory_space=pltpu.VMEM)] * 5,
        out_specs=pl.BlockSpec(memory_space=pltpu.VMEM),
        scratch_shapes=[
            pltpu.VMEM((N_DEV, e_loc, d, h), jnp.bfloat16),
            pltpu.SemaphoreType.DMA((7,)),
            pltpu.SemaphoreType.DMA((7,)),
        ],
        compiler_params=pltpu.CompilerParams(collective_id=0),
    )(x, router_W, route_idx, expert_W, shared_W)


# device time: 47946 ns/iter; 3.8097x vs baseline; 1.0901x over previous
import jax
import jax.numpy as jnp
from jax import lax
from jax.experimental import pallas as pl
from jax.experimental.pallas import tpu as pltpu

N_DEV = 8

_SENDS = (
    (0, 0, slice(None), 1, 1),
    (1, 0, slice(None), 3, 3),
    (2, 0, slice(None), 4, 4),
    (3, 1, slice(None), 2, 3),
    (4, 3, slice(None), 7, 4),
    (5, 4, slice(None), 5, 1),
    (6, 7, slice(0, 2), 6, 1),
    (7, 5, slice(2, 4), 6, 3),
)


def kernel(x, router_W, route_idx, expert_W, shared_W):
    n_tok, d = x.shape
    e_loc, _, h = expert_W.shape

    def body(x_ref, rW_ref, idx_ref, eW_ref, sW_ref, out_ref,
             chunks_ref, send_sems, recv_sems):
        my = lax.axis_index("i")
        nbr = {m: jnp.bitwise_xor(my, m) for m in (1, 3, 4)}

        chunks_ref[0] = eW_ref[...].astype(jnp.bfloat16)

        barrier_sem = pltpu.get_barrier_semaphore()
        for m in (1, 3, 4):
            pl.semaphore_signal(
                barrier_sem, inc=1,
                device_id=(nbr[m],), device_id_type=pl.DeviceIdType.MESH,
            )
        pl.semaphore_wait(barrier_sem, 3)

        def make_send(k):
            _, src, es, dst, m = _SENDS[k]
            return pltpu.make_async_remote_copy(
                src_ref=chunks_ref.at[src, es],
                dst_ref=chunks_ref.at[dst, es],
                send_sem=send_sems.at[k],
                recv_sem=recv_sems.at[k],
                device_id=(nbr[m],),
                device_id_type=pl.DeviceIdType.MESH,
            )

        rdmas = {}
        for k in (0, 1, 2):
            rdmas[k] = make_send(k)
            rdmas[k].start()

        xv = x_ref[...]
        xb = xv.astype(jnp.bfloat16)
        idx = idx_ref[...]

        scores = jnp.dot(xv, rW_ref[...], preferred_element_type=jnp.float32)
        smax = jnp.max(scores, axis=1, keepdims=True)
        probs = jnp.exp(scores - smax)
        probs = probs / jnp.sum(probs, axis=1, keepdims=True)
        eids = lax.broadcasted_iota(jnp.int32, scores.shape, 1)
        p = jnp.sum(jnp.where(eids == idx, probs, 0.0), axis=1, keepdims=True)

        acc = jnp.dot(xv, sW_ref[...], preferred_element_type=jnp.float32)

        def add_chunk(acc, slot):
            origin = jnp.bitwise_xor(my, slot)
            for j in range(e_loc):
                e = origin * e_loc + j
                gate = (p * (idx == e).astype(jnp.float32)).astype(jnp.bfloat16)
                acc = acc + jnp.dot(
                    xb * gate, chunks_ref[slot, j],
                    preferred_element_type=jnp.float32,
                )
            return acc

        acc = add_chunk(acc, 0)

        def step(k, fwd):
            rdmas[k].wait()
            for kk in fwd:
                rdmas[kk] = make_send(kk)
                rdmas[kk].start()

        step(0, (3,));  acc = add_chunk(acc, 1)
        step(1, (4,));  acc = add_chunk(acc, 3)
        step(2, (5,));  acc = add_chunk(acc, 4)
        step(4, (6,));  acc = add_chunk(acc, 7)
        step(5, (7,));  acc = add_chunk(acc, 5)
        step(3, ());    acc = add_chunk(acc, 2)
        step(6, ())
        step(7, ());    acc = add_chunk(acc, 6)

        out_ref[...] = acc

    return pl.pallas_call(
        body,
        out_shape=jax.ShapeDtypeStruct((n_tok, h), jnp.float32),
        in_specs=[pl.BlockSpec(memory_space=pltpu.VMEM)] * 5,
        out_specs=pl.BlockSpec(memory_space=pltpu.VMEM),
        scratch_shapes=[
            pltpu.VMEM((N_DEV, e_loc, d, h), jnp.bfloat16),
            pltpu.SemaphoreType.DMA((8,)),
            pltpu.SemaphoreType.DMA((8,)),
        ],
        compiler_params=pltpu.CompilerParams(collective_id=0),
    )(x, router_W, route_idx, expert_W, shared_W)
